# baseline (speedup 1.0000x reference)
import jax
import jax.numpy as jnp
from jax import lax
from jax.experimental import pallas as pl
from jax.experimental.pallas import tpu as pltpu
from jax.experimental.pallas import tpu_sc as plsc

B = 4096
S = 200
D = 64
C = 4
V = 1000000
INV_S = 1.0 / S

NC = 2
NS = 16
LANES = 16
BPT = B // NS
NG = BPT // LANES
GIDX = LANES * S

BK = 65536


def _proj_body(w_ref, b_ref, x_ref, o0_ref, o1_ref):
    p = lax.dot_general(w_ref[...], x_ref[...], (((0,), (0,)), ((), ())),
                        preferred_element_type=jnp.float32)
    q = (p + b_ref[...]) * INV_S
    u = lax.bitcast_convert_type(q.astype(jnp.bfloat16), jnp.uint16)
    lo = u[0:2].astype(jnp.uint32)
    hi = u[2:4].astype(jnp.uint32)
    w = lax.bitcast_convert_type((hi << 16) | lo, jnp.int32)
    o0_ref[...] = w[0]
    o1_ref[...] = w[1]


def _project(tableT, Wp, bp):
    nblk = (V + BK - 1) // BK
    return pl.pallas_call(
        _proj_body,
        grid=(nblk,),
        in_specs=[
            pl.BlockSpec((D, C), lambda i: (0, 0)),
            pl.BlockSpec((C, 1), lambda i: (0, 0)),
            pl.BlockSpec((D, BK), lambda i: (0, i)),
        ],
        out_specs=[
            pl.BlockSpec((BK,), lambda i: (i,)),
            pl.BlockSpec((BK,), lambda i: (i,)),
        ],
        out_shape=[
            jax.ShapeDtypeStruct((V,), jnp.int32),
            jax.ShapeDtypeStruct((V,), jnp.int32),
        ],
    )(Wp, bp, tableT)


def _pool_body(ids_hbm, qp0_hbm, qp1_hbm, out_hbm,
               idx_v, idxt0, idxt1, d0, d1, outlo, outhi, plane, s0, s1):
    c = lax.axis_index("c")
    sid = lax.axis_index("s")
    base = sid * BPT

    idxts = (idxt0, idxt1)
    dests = (d0, d1)
    sems = (s0, s1)

    pltpu.sync_copy(ids_hbm.at[pl.ds(base, BPT)], idx_v)

    @pl.when(jnp.logical_and(sid == 0, c == 0))
    def _():
        pltpu.sync_copy(qp0_hbm, plane)

    @pl.when(jnp.logical_and(sid == 0, c == 1))
    def _():
        pltpu.sync_copy(qp1_hbm, plane)

    plsc.subcore_barrier()

    def build(buf, g):
        rows = g * LANES + lax.iota(jnp.int32, 16)

        def bs(s, _):
            v = plsc.load_gather(idx_v, [rows, jnp.full((16,), s, jnp.int32)])
            idxts[buf][pl.ds(s * LANES, LANES)] = v
            return 0

        lax.fori_loop(0, S, bs, 0)

    def fire(buf):
        pltpu.async_copy(plane.at[idxts[buf]], dests[buf], sems[buf])

    def drain(buf):
        pltpu.make_async_copy(plane.at[idxts[buf]], dests[buf],
                              sems[buf]).wait()

    mask_hi = jnp.full((LANES,), -65536, jnp.int32)

    def consume(g, buf):
        def sb(s, acc):
            alo, ahi = acc
            w = dests[buf][pl.ds(s * LANES, LANES)]
            flo = plsc.bitcast(lax.shift_left(w, 16), jnp.float32)
            fhi = plsc.bitcast(lax.bitwise_and(w, mask_hi), jnp.float32)
            return (alo + flo, ahi + fhi)

        zero = jnp.zeros((LANES,), jnp.float32)
        alo, ahi = lax.fori_loop(0, S, sb, (zero, zero), unroll=4)
        outlo[pl.ds(g * LANES, LANES)] = alo
        outhi[pl.ds(g * LANES, LANES)] = ahi

    build(0, 0)
    fire(0)
    for g in range(NG):
        buf = g & 1
        if g + 1 < NG:
            build(1 - buf, g + 1)
            fire(1 - buf)
        drain(buf)
        consume(g, buf)

    pltpu.sync_copy(outlo, out_hbm.at[2 * c, pl.ds(base, BPT)])
    pltpu.sync_copy(outhi, out_hbm.at[2 * c + 1, pl.ds(base, BPT)])


@jax.jit
def _sc_pool(ids, qp0, qp1):
    mesh = plsc.VectorSubcoreMesh(core_axis_name="c", subcore_axis_name="s",
                                  num_cores=NC, num_subcores=NS)
    return pl.kernel(
        _pool_body,
        out_type=jax.ShapeDtypeStruct((C, B), jnp.float32),
        mesh=mesh,
        compiler_params=pltpu.CompilerParams(use_tc_tiling_on_sc=False,
                                             needs_layout_passes=False),
        scratch_types=[
            pltpu.VMEM((BPT, S), jnp.int32),
            pltpu.VMEM((GIDX,), jnp.int32),
            pltpu.VMEM((GIDX,), jnp.int32),
            pltpu.VMEM((GIDX,), jnp.int32),
            pltpu.VMEM((GIDX,), jnp.int32),
            pltpu.VMEM((BPT,), jnp.float32),
            pltpu.VMEM((BPT,), jnp.float32),
            pltpu.VMEM_SHARED((V,), jnp.int32),
            pltpu.SemaphoreType.DMA,
            pltpu.SemaphoreType.DMA,
        ],
    )(ids, qp0, qp1)


def kernel(input_ids, table, W, b):
    perm = jnp.array([0, 2, 1, 3], dtype=jnp.int32)
    qp0, qp1 = _project(table.T, W[:, perm], b[perm].reshape(C, 1))
    out = _sc_pool(input_ids, qp0, qp1)
    return out.T


# speedup vs baseline: 1.1158x; 1.1158x over previous
import jax
import jax.numpy as jnp
from jax import lax
from jax.experimental import pallas as pl
from jax.experimental.pallas import tpu as pltpu
from jax.experimental.pallas import tpu_sc as plsc

B = 4096
S = 200
D = 64
C = 4
V = 1000000
INV_S = 1.0 / S

NC = 2
NS = 16
LANES = 16
BPT = B // NS
NG = BPT // LANES
GIDX = LANES * S

BK = 32768


def _proj_body(w_ref, b_ref, x_ref, o0_ref, o1_ref):
    p = lax.dot_general(w_ref[...], x_ref[...], (((0,), (0,)), ((), ())),
                        preferred_element_type=jnp.float32)
    q = (p + b_ref[...]) * INV_S
    u = lax.bitcast_convert_type(q.astype(jnp.bfloat16), jnp.uint16)
    lo = u[0:2].astype(jnp.uint32)
    hi = u[2:4].astype(jnp.uint32)
    w = lax.bitcast_convert_type((hi << 16) | lo, jnp.int32)
    o0_ref[...] = w[0]
    o1_ref[...] = w[1]


def _project(tableT, Wp, bp):
    nblk = (V + BK - 1) // BK
    return pl.pallas_call(
        _proj_body,
        grid=(nblk,),
        in_specs=[
            pl.BlockSpec((D, C), lambda i: (0, 0)),
            pl.BlockSpec((C, 1), lambda i: (0, 0)),
            pl.BlockSpec((D, BK), lambda i: (0, i)),
        ],
        out_specs=[
            pl.BlockSpec((BK,), lambda i: (i,)),
            pl.BlockSpec((BK,), lambda i: (i,)),
        ],
        out_shape=[
            jax.ShapeDtypeStruct((V,), jnp.int32),
            jax.ShapeDtypeStruct((V,), jnp.int32),
        ],
    )(Wp, bp, tableT)


def _pool_body(ids4_hbm, qp0_hbm, qp1_hbm, out_hbm,
               idx_v, idxt0, idxt1, d0, d1, outlo, outhi, plane, s0, s1):
    c = lax.axis_index("c")
    sid = lax.axis_index("s")
    base = sid * BPT

    idxts = (idxt0, idxt1)
    dests = (d0, d1)
    sems = (s0, s1)

    pltpu.sync_copy(ids4_hbm.at[:, pl.ds(2 * sid, 2)], idx_v)

    @pl.when(jnp.logical_and(sid == 0, c == 0))
    def _():
        pltpu.sync_copy(qp0_hbm, plane)

    @pl.when(jnp.logical_and(sid == 0, c == 1))
    def _():
        pltpu.sync_copy(qp1_hbm, plane)

    plsc.subcore_barrier()

    def build(buf, g):
        bt = g // 8
        lane_off = (g % 8) * LANES

        def bs(s, _):
            v = idx_v[s // 8, bt, s % 8, pl.ds(lane_off, LANES)]
            idxts[buf][pl.ds(s * LANES, LANES)] = v
            return 0

        lax.fori_loop(0, S, bs, 0)

    def fire(buf):
        pltpu.async_copy(plane.at[idxts[buf]], dests[buf], sems[buf])

    def drain(buf):
        pltpu.make_async_copy(plane.at[idxts[buf]], dests[buf],
                              sems[buf]).wait()

    mask_hi = jnp.full((LANES,), -65536, jnp.int32)

    def consume(g, buf):
        def sb(s, acc):
            alo, ahi = acc
            w = dests[buf][pl.ds(s * LANES, LANES)]
            flo = plsc.bitcast(lax.shift_left(w, 16), jnp.float32)
            fhi = plsc.bitcast(lax.bitwise_and(w, mask_hi), jnp.float32)
            return (alo + flo, ahi + fhi)

        zero = jnp.zeros((LANES,), jnp.float32)
        alo, ahi = lax.fori_loop(0, S, sb, (zero, zero), unroll=4)
        outlo[pl.ds(g * LANES, LANES)] = alo
        outhi[pl.ds(g * LANES, LANES)] = ahi

    build(0, 0)
    fire(0)
    for g in range(NG):
        buf = g & 1
        if g + 1 < NG:
            build(1 - buf, g + 1)
            fire(1 - buf)
        drain(buf)
        consume(g, buf)

    pltpu.sync_copy(outlo, out_hbm.at[2 * c, pl.ds(base, BPT)])
    pltpu.sync_copy(outhi, out_hbm.at[2 * c + 1, pl.ds(base, BPT)])


@jax.jit
def _sc_pool(ids, qp0, qp1):
    mesh = plsc.VectorSubcoreMesh(core_axis_name="c", subcore_axis_name="s",
                                  num_cores=NC, num_subcores=NS)
    return pl.kernel(
        _pool_body,
        out_type=jax.ShapeDtypeStruct((C, B), jnp.float32),
        mesh=mesh,
        compiler_params=pltpu.CompilerParams(use_tc_tiling_on_sc=False,
                                             needs_layout_passes=False),
        scratch_types=[
            pltpu.VMEM((S // 8, 2, 8, 128), jnp.int32),
            pltpu.VMEM((GIDX,), jnp.int32),
            pltpu.VMEM((GIDX,), jnp.int32),
            pltpu.VMEM((GIDX,), jnp.int32),
            pltpu.VMEM((GIDX,), jnp.int32),
            pltpu.VMEM((BPT,), jnp.float32),
            pltpu.VMEM((BPT,), jnp.float32),
            pltpu.VMEM_SHARED((V,), jnp.int32),
            pltpu.SemaphoreType.DMA,
            pltpu.SemaphoreType.DMA,
        ],
    )(ids, qp0, qp1)


def kernel(input_ids, table, W, b):
    perm = jnp.array([0, 2, 1, 3], dtype=jnp.int32)
    qp0, qp1 = _project(table.T, W[:, perm], b[perm].reshape(C, 1))
    ids4 = jnp.transpose(input_ids.reshape(32, 128, S // 8, 8), (2, 0, 3, 1))
    out = _sc_pool(ids4, qp0, qp1)
    return out.T
